# baseline (speedup 1.0000x reference)
import functools

import jax
import jax.numpy as jnp
from jax import lax
from jax.experimental import pallas as pl
from jax.experimental.pallas import tpu as pltpu
from jax.experimental.pallas import tpu_sc as plsc

B = 64
S_IN = 64
S_OUT = 256
D = 768
LANES = 16
NC, NS = 2, 16
NW = NC * NS
B_PER_W = B // NW
BLK = 32

_mesh = plsc.VectorSubcoreMesh(core_axis_name="c", subcore_axis_name="s")


@functools.partial(
    pl.kernel,
    out_type=jax.ShapeDtypeStruct((B * S_OUT, D), jnp.float32),
    mesh=_mesh,
    scratch_types=[
        pltpu.VMEM((BLK, D), jnp.float32),
        pltpu.VMEM((S_IN, D), jnp.float32),
        pltpu.VMEM((S_IN, D), jnp.float32),
        pltpu.VMEM_SHARED((BLK, D), jnp.float32),
        pltpu.SemaphoreType.DMA,
        pltpu.SemaphoreType.DMA,
    ],
)
def _mask_token_sc(in_hbm, mst_hbm, out_hbm, blk, stage0, stage1, sblk, gsem, wsem):
    wid = lax.axis_index("s") * NC + lax.axis_index("c")
    b0 = wid * B_PER_W
    stages = (stage0, stage1)

    gathers = [
        pltpu.async_copy(in_hbm.at[pl.ds((b0 + k) * S_IN, S_IN)], stages[k], gsem)
        for k in range(B_PER_W)
    ]

    pltpu.sync_copy(mst_hbm, blk.at[0])

    def rep(r, carry):
        for j in range(D // LANES):
            blk[r, pl.ds(j * LANES, LANES)] = blk[0, pl.ds(j * LANES, LANES)]
        return carry

    lax.fori_loop(1, BLK, rep, 0)

    @pl.when(lax.axis_index("s") == 0)
    def _():
        pltpu.sync_copy(blk, sblk)

    plsc.subcore_barrier()

    writes = []
    for k in range(B_PER_W):
        src = blk if k == 0 else sblk
        out_base = (b0 + k) * S_OUT
        for c in range(128 // BLK):
            writes.append(
                pltpu.async_copy(src, out_hbm.at[pl.ds(out_base + c * BLK, BLK)], wsem)
            )
        for c in range(64 // BLK):
            writes.append(
                pltpu.async_copy(src, out_hbm.at[pl.ds(out_base + 192 + c * BLK, BLK)], wsem)
            )

    for k in range(B_PER_W):
        gathers[k].wait()
        writes.append(
            pltpu.async_copy(
                stages[k], out_hbm.at[pl.ds((b0 + k) * S_OUT + 128, S_IN)], wsem
            )
        )
    for w in writes:
        w.wait()


def kernel(inputs, mst):
    out = _mask_token_sc(
        inputs.reshape(B * S_IN, D),
        mst.astype(inputs.dtype).reshape(D),
    )
    return out.reshape(B, S_OUT, D)


# speedup vs baseline: 1.0235x; 1.0235x over previous
import functools

import jax
import jax.numpy as jnp
from jax import lax
from jax.experimental import pallas as pl
from jax.experimental.pallas import tpu as pltpu
from jax.experimental.pallas import tpu_sc as plsc

B = 64
S_IN = 64
S_OUT = 256
D = 768
LANES = 16
NC, NS = 2, 16
NW = NC * NS
B_PER_W = B // NW
BLK = 32

_mesh = plsc.VectorSubcoreMesh(core_axis_name="c", subcore_axis_name="s")


@functools.partial(
    pl.kernel,
    out_type=jax.ShapeDtypeStruct((B * S_OUT, D), jnp.float32),
    mesh=_mesh,
    scratch_types=[
        pltpu.VMEM((BLK, D), jnp.float32),
        pltpu.VMEM((S_IN, D), jnp.float32),
        pltpu.VMEM((S_IN, D), jnp.float32),
        pltpu.SemaphoreType.DMA,
        pltpu.SemaphoreType.DMA,
    ],
)
def _mask_token_sc(in_hbm, mst_hbm, out_hbm, blk, stage0, stage1, gsem, wsem):
    wid = lax.axis_index("s") * NC + lax.axis_index("c")
    b0 = wid * B_PER_W
    stages = (stage0, stage1)

    gathers = [
        pltpu.async_copy(in_hbm.at[pl.ds((b0 + k) * S_IN, S_IN)], stages[k], gsem)
        for k in range(B_PER_W)
    ]

    pltpu.sync_copy(mst_hbm, blk.at[0])

    def rep(r, carry):
        for j in range(D // LANES):
            blk[r, pl.ds(j * LANES, LANES)] = blk[0, pl.ds(j * LANES, LANES)]
        return carry

    lax.fori_loop(1, BLK, rep, 0)

    writes = []
    for k in range(B_PER_W):
        out_base = (b0 + k) * S_OUT
        for c in range(128 // BLK):
            writes.append(
                pltpu.async_copy(blk, out_hbm.at[pl.ds(out_base + c * BLK, BLK)], wsem)
            )
        for c in range(64 // BLK):
            writes.append(
                pltpu.async_copy(blk, out_hbm.at[pl.ds(out_base + 192 + c * BLK, BLK)], wsem)
            )

    for k in range(B_PER_W):
        gathers[k].wait()
        writes.append(
            pltpu.async_copy(
                stages[k], out_hbm.at[pl.ds((b0 + k) * S_OUT + 128, S_IN)], wsem
            )
        )
    for w in writes:
        w.wait()


def kernel(inputs, mst):
    out = _mask_token_sc(
        inputs.reshape(B * S_IN, D),
        mst.astype(inputs.dtype).reshape(D),
    )
    return out.reshape(B, S_OUT, D)
